# Initial kernel scaffold; baseline (speedup 1.0000x reference)
import jax
import jax.numpy as jnp
from jax.experimental import pallas as pl


def kernel(x):
    raise NotImplementedError("write your pallas kernel here")


# speedup vs baseline: 2.6818x; 2.6818x over previous
import jax
import jax.numpy as jnp
from jax.experimental import pallas as pl

_ROWS_PER_BLOCK = 8
_N = 32768


def _top5_round_body(x_ref, o_ref):
    x = x_ref[...]
    col = jax.lax.broadcasted_iota(jnp.int32, x.shape, 1)
    work = x
    selected = jnp.zeros(x.shape, jnp.bool_)
    for _ in range(5):
        m = jnp.max(work, axis=1, keepdims=True)
        eq = work == m
        first_col = jnp.min(jnp.where(eq, col, _N), axis=1, keepdims=True)
        first = col == first_col
        selected = selected | first
        work = jnp.where(first, -jnp.inf, work)
    o_ref[...] = jnp.where(selected, jnp.round(x), 0.0)


def kernel(x):
    rows, n = x.shape
    grid = (rows // _ROWS_PER_BLOCK,)
    return pl.pallas_call(
        _top5_round_body,
        grid=grid,
        in_specs=[pl.BlockSpec((_ROWS_PER_BLOCK, n), lambda i: (i, 0))],
        out_specs=pl.BlockSpec((_ROWS_PER_BLOCK, n), lambda i: (i, 0)),
        out_shape=jax.ShapeDtypeStruct(x.shape, x.dtype),
    )(x)


# speedup vs baseline: 5.1037x; 1.9031x over previous
import jax
import jax.numpy as jnp
from jax.experimental import pallas as pl

_ROWS = 8
_N = 32768
_CHUNK = 128
_NCHUNKS = _N // _CHUNK
_NEG = float("-inf")


def _exact_top5_mask(x):
    col = jax.lax.broadcasted_iota(jnp.int32, x.shape, 1)
    work = x
    for _ in range(5):
        m = jnp.max(work, axis=1, keepdims=True)
        eq = work == m
        first_col = jnp.min(jnp.where(eq, col, _N), axis=1, keepdims=True)
        work = jnp.where(col == first_col, _NEG, work)
    return work == _NEG


def _top5_round_body(x_ref, o_ref):
    x = x_ref[...]

    t1 = t2 = t3 = t4 = t5 = jnp.full((_ROWS, _CHUNK), _NEG)
    for k in range(_NCHUNKS):
        v = x[:, k * _CHUNK:(k + 1) * _CHUNK]
        m1 = jnp.maximum(t1, v)
        r1 = jnp.minimum(t1, v)
        m2 = jnp.maximum(t2, r1)
        r2 = jnp.minimum(t2, r1)
        m3 = jnp.maximum(t3, r2)
        r3 = jnp.minimum(t3, r2)
        m4 = jnp.maximum(t4, r3)
        r4 = jnp.minimum(t4, r3)
        m5 = jnp.maximum(t5, r4)
        t1, t2, t3, t4, t5 = m1, m2, m3, m4, m5

    cand = jnp.concatenate([t1, t2, t3, t4, t5], axis=1)
    ccol = jax.lax.broadcasted_iota(jnp.int32, cand.shape, 1)
    vals = []
    for _ in range(5):
        m = jnp.max(cand, axis=1, keepdims=True)
        vals.append(m)
        first_col = jnp.min(
            jnp.where(cand == m, ccol, cand.shape[1]), axis=1, keepdims=True)
        cand = jnp.where(ccol == first_col, _NEG, cand)
    v4, v5 = vals[3], vals[4]

    fast = jnp.all(v4 > v5)

    @pl.when(fast)
    def _():
        col = jax.lax.broadcasted_iota(jnp.int32, x.shape, 1)
        first_eq = jnp.min(jnp.where(x == v5, col, _N), axis=1, keepdims=True)
        keep = (x > v5) | (col == first_eq)
        o_ref[...] = jnp.where(keep, jnp.round(x), 0.0)

    @pl.when(jnp.logical_not(fast))
    def _():
        o_ref[...] = jnp.where(_exact_top5_mask(x), jnp.round(x), 0.0)


def kernel(x):
    rows, n = x.shape
    grid = (rows // _ROWS,)
    return pl.pallas_call(
        _top5_round_body,
        grid=grid,
        in_specs=[pl.BlockSpec((_ROWS, n), lambda i: (i, 0))],
        out_specs=pl.BlockSpec((_ROWS, n), lambda i: (i, 0)),
        out_shape=jax.ShapeDtypeStruct(x.shape, x.dtype),
    )(x)
